# baseline (speedup 1.0000x reference)
import functools

import jax
import jax.numpy as jnp
from jax import lax
from jax.experimental import pallas as pl
from jax.experimental.pallas import tpu as pltpu
from jax.experimental.pallas import tpu_sc as plsc

N_NODES = 10000
N_EDGES = 160000
D_FEAT = 256
D_EDGE = 16
D_OUT = 256
D_HALF = D_FEAT // 2

NC, NS, LANES = 2, 16, 16
NW = NC * NS
CHUNK = 40
NBUF = 5
PREF = 3
M_HI = -65536
R_HALF = 0x8000

CHUNKS_1 = 55
CHUNKS_2 = 70
E_SLICE1 = NW * CHUNKS_1 * CHUNK
E_SLICE2 = NW * CHUNKS_2 * CHUNK
M_FIN = 3200


def _make_gather_sum(start, n_chunks):
    epw = n_chunks * CHUNK

    def body(xs_hbm, xt_hbm, iflat_hbm, out_hbm,
             idx, bufa, bufb, sem_i, sem_ga, sem_gb, sem_w):
        wid = lax.axis_index("s") * NC + lax.axis_index("c")

        def idx_copy_parts(chunk, b):
            base = start + wid * epw + chunk * CHUNK
            return (
                pltpu.make_async_copy(iflat_hbm.at[pl.ds(base, CHUNK)],
                                      idx.at[b, 0], sem_i.at[b]),
                pltpu.make_async_copy(
                    iflat_hbm.at[pl.ds(N_EDGES + base, CHUNK)],
                    idx.at[b, 1], sem_i.at[b]),
            )

        def idx_start(chunk, b):
            for part in idx_copy_parts(chunk, b):
                part.start()

        def idx_wait(chunk, b):
            for part in idx_copy_parts(chunk, b):
                part.wait()

        def gather_a(chunk, b):
            return pltpu.make_async_copy(xs_hbm.at[idx.at[b, 0]], bufa.at[b],
                                         sem_ga.at[b])

        def gather_b(chunk, b):
            return pltpu.make_async_copy(xt_hbm.at[idx.at[b, 1]], bufb.at[b],
                                         sem_gb.at[b])

        def out_copy(chunk, b):
            return pltpu.make_async_copy(
                bufa.at[b],
                out_hbm.at[pl.ds(wid * epw + chunk * CHUNK, CHUNK)],
                sem_w.at[b])

        for c in range(PREF + 1):
            idx_start(c, c)
        for c in range(PREF):
            idx_wait(c, c)
            gather_a(c, c).start()
            gather_b(c, c).start()

        @pl.loop(0, n_chunks, step=NBUF)
        def outer(base_chunk):
            for b in range(NBUF):
                chunk = base_chunk + b
                gather_a(chunk, b).wait()
                gather_b(chunk, b).wait()

                def row_body(e, carry):
                    for j in range(D_HALF // LANES):
                        sl = pl.ds(j * LANES, LANES)
                        va = bufa[b, e, sl]
                        vb = bufb[b, e, sl]
                        lo = (lax.bitcast_convert_type(va << 16, jnp.float32)
                              + lax.bitcast_convert_type(vb << 16,
                                                         jnp.float32))
                        hi = (lax.bitcast_convert_type(va & M_HI, jnp.float32)
                              + lax.bitcast_convert_type(vb & M_HI,
                                                         jnp.float32))
                        lo_i = lax.shift_right_logical(
                            lax.bitcast_convert_type(lo, jnp.int32) + R_HALF,
                            16)
                        hi_i = (lax.bitcast_convert_type(hi, jnp.int32)
                                + R_HALF) & M_HI
                        bufa[b, e, sl] = lo_i | hi_i
                    return carry

                lax.fori_loop(0, CHUNK, row_body, 0)
                out_copy(chunk, b).start()

                nb = (b + PREF) % NBUF
                ib = (b + PREF + 1) % NBUF

                @pl.when(chunk + PREF + 1 < n_chunks)
                def _prefetch_idx():
                    idx_start(chunk + PREF + 1, ib)

                @pl.when((chunk + PREF < n_chunks) & (chunk >= NBUF - PREF))
                def _wait_writeout():
                    out_copy(chunk - (NBUF - PREF), nb).wait()

                @pl.when(chunk + PREF < n_chunks)
                def _prefetch_gathers():
                    idx_wait(chunk + PREF, nb)
                    gather_a(chunk + PREF, nb).start()
                    gather_b(chunk + PREF, nb).start()

        for b in range(NBUF):
            out_copy(n_chunks - NBUF + b, b).wait()

    return pl.kernel(
        body,
        out_type=jax.ShapeDtypeStruct((NW * epw, D_HALF), jnp.int32),
        mesh=plsc.VectorSubcoreMesh(core_axis_name="c", subcore_axis_name="s"),
        scratch_types=[
            pltpu.VMEM((NBUF, 2, CHUNK), jnp.int32),
            pltpu.VMEM((NBUF, CHUNK, D_HALF), jnp.int32),
            pltpu.VMEM((NBUF, CHUNK, D_HALF), jnp.int32),
            pltpu.SemaphoreType.DMA((NBUF,)),
            pltpu.SemaphoreType.DMA((NBUF,)),
            pltpu.SemaphoreType.DMA((NBUF,)),
            pltpu.SemaphoreType.DMA((NBUF,)),
        ],
    )


_gather_sum_1 = _make_gather_sum(0, CHUNKS_1)
_gather_sum_2 = _make_gather_sum(E_SLICE1, CHUNKS_2)


def _pack_halves(y):
    ylo = lax.bitcast_convert_type(y[:, :D_HALF], jnp.int32)
    yhi = lax.bitcast_convert_type(y[:, D_HALF:], jnp.int32)
    lo16 = lax.shift_right_logical(ylo + R_HALF, 16)
    hi16 = (yhi + R_HALF) & M_HI
    return lo16 | hi16


def _project_body(x_ref, ws_ref, wt_ref, xs_ref, xt_ref):
    xb = x_ref[...]
    xs_ref[...] = _pack_halves(
        jnp.dot(xb, ws_ref[...], preferred_element_type=jnp.float32))
    xt_ref[...] = _pack_halves(
        jnp.dot(xb, wt_ref[...], preferred_element_type=jnp.float32))


def _project(x, ws, wt):
    m_blk = 1000
    grid = (N_NODES // m_blk,)
    return pl.pallas_call(
        _project_body,
        grid=grid,
        in_specs=[
            pl.BlockSpec((m_blk, D_FEAT), lambda i: (i, 0)),
            pl.BlockSpec((D_FEAT, D_FEAT), lambda i: (0, 0)),
            pl.BlockSpec((D_FEAT, D_FEAT), lambda i: (0, 0)),
        ],
        out_specs=[
            pl.BlockSpec((m_blk, D_HALF), lambda i: (i, 0)),
            pl.BlockSpec((m_blk, D_HALF), lambda i: (i, 0)),
        ],
        out_shape=[
            jax.ShapeDtypeStruct((N_NODES, D_HALF), jnp.int32),
            jax.ShapeDtypeStruct((N_NODES, D_HALF), jnp.int32),
        ],
    )(x, ws, wt)


def _final_body(ef_ref, s_ref, we_ref, b_ref, o_ref):
    acc = jnp.dot(ef_ref[...], we_ref[...], preferred_element_type=jnp.float32)
    si = s_ref[...]
    lo = lax.bitcast_convert_type(si << 16, jnp.float32)
    hi = lax.bitcast_convert_type(si & M_HI, jnp.float32)
    s = jnp.concatenate([lo, hi], axis=1)
    o_ref[...] = jnp.maximum(acc + s + b_ref[...], 0.0)


def _final_slice1(edge_feat, s, we, b2d):
    grid = (E_SLICE1 // M_FIN,)
    return pl.pallas_call(
        _final_body,
        grid=grid,
        in_specs=[
            pl.BlockSpec((M_FIN, D_EDGE), lambda i: (i, 0)),
            pl.BlockSpec((M_FIN, D_HALF), lambda i: (i, 0)),
            pl.BlockSpec((D_EDGE, D_OUT), lambda i: (0, 0)),
            pl.BlockSpec((1, D_OUT), lambda i: (0, 0)),
        ],
        out_specs=pl.BlockSpec((M_FIN, D_OUT), lambda i: (i, 0)),
        out_shape=jax.ShapeDtypeStruct((N_EDGES, D_OUT), jnp.float32),
    )(edge_feat, s, we, b2d)


def _final_body2(_, ef_ref, s_ref, we_ref, b_ref, o_ref):
    _final_body(ef_ref, s_ref, we_ref, b_ref, o_ref)


def _final_slice2(out1, edge_feat, s, we, b2d):
    grid = (E_SLICE2 // M_FIN,)
    off = E_SLICE1 // M_FIN
    return pl.pallas_call(
        _final_body2,
        grid=grid,
        in_specs=[
            pl.BlockSpec(memory_space=pl.ANY),
            pl.BlockSpec((M_FIN, D_EDGE), lambda i: (i + off, 0)),
            pl.BlockSpec((M_FIN, D_HALF), lambda i: (i, 0)),
            pl.BlockSpec((D_EDGE, D_OUT), lambda i: (0, 0)),
            pl.BlockSpec((1, D_OUT), lambda i: (0, 0)),
        ],
        out_specs=pl.BlockSpec((M_FIN, D_OUT), lambda i: (i + off, 0)),
        out_shape=jax.ShapeDtypeStruct((N_EDGES, D_OUT), jnp.float32),
        input_output_aliases={0: 0},
    )(out1, edge_feat, s, we, b2d)


def kernel(x, edge_feat, edge_index, W, b):
    we = W[:D_EDGE]
    ws = W[D_EDGE:D_EDGE + D_FEAT]
    wt = W[D_EDGE + D_FEAT:]
    b2d = b.reshape(1, D_OUT)
    xs, xt = _project(x, ws, wt)
    iflat = edge_index.reshape(2 * N_EDGES)
    s1 = _gather_sum_1(xs, xt, iflat)
    s2 = _gather_sum_2(xs, xt, iflat)
    out1 = _final_slice1(edge_feat, s1, we, b2d)
    return _final_slice2(out1, edge_feat, s2, we, b2d)


# speedup vs baseline: 1.0100x; 1.0100x over previous
import functools

import jax
import jax.numpy as jnp
from jax import lax
from jax.experimental import pallas as pl
from jax.experimental.pallas import tpu as pltpu
from jax.experimental.pallas import tpu_sc as plsc

N_NODES = 10000
N_EDGES = 160000
D_FEAT = 256
D_EDGE = 16
D_OUT = 256
D_HALF = D_FEAT // 2

NC, NS, LANES = 2, 16, 16
NW = NC * NS
CHUNK = 40
NBUF = 5
PREF = 3
M_HI = -65536
R_HALF = 0x8000

CHUNKS_1 = 60
CHUNKS_2 = 65
E_SLICE1 = NW * CHUNKS_1 * CHUNK
E_SLICE2 = NW * CHUNKS_2 * CHUNK
M_FIN = 3200


def _make_gather_sum(start, n_chunks):
    epw = n_chunks * CHUNK

    def body(xs_hbm, xt_hbm, iflat_hbm, out_hbm,
             idx, bufa, bufb, sem_i, sem_ga, sem_gb, sem_w):
        wid = lax.axis_index("s") * NC + lax.axis_index("c")

        def idx_copy_parts(chunk, b):
            base = start + wid * epw + chunk * CHUNK
            return (
                pltpu.make_async_copy(iflat_hbm.at[pl.ds(base, CHUNK)],
                                      idx.at[b, 0], sem_i.at[b]),
                pltpu.make_async_copy(
                    iflat_hbm.at[pl.ds(N_EDGES + base, CHUNK)],
                    idx.at[b, 1], sem_i.at[b]),
            )

        def idx_start(chunk, b):
            for part in idx_copy_parts(chunk, b):
                part.start()

        def idx_wait(chunk, b):
            for part in idx_copy_parts(chunk, b):
                part.wait()

        def gather_a(chunk, b):
            return pltpu.make_async_copy(xs_hbm.at[idx.at[b, 0]], bufa.at[b],
                                         sem_ga.at[b])

        def gather_b(chunk, b):
            return pltpu.make_async_copy(xt_hbm.at[idx.at[b, 1]], bufb.at[b],
                                         sem_gb.at[b])

        def out_copy(chunk, b):
            return pltpu.make_async_copy(
                bufa.at[b],
                out_hbm.at[pl.ds(wid * epw + chunk * CHUNK, CHUNK)],
                sem_w.at[b])

        for c in range(PREF + 1):
            idx_start(c, c)
        for c in range(PREF):
            idx_wait(c, c)
            gather_a(c, c).start()
            gather_b(c, c).start()

        @pl.loop(0, n_chunks, step=NBUF)
        def outer(base_chunk):
            for b in range(NBUF):
                chunk = base_chunk + b
                gather_a(chunk, b).wait()
                gather_b(chunk, b).wait()

                def row_body(e, carry):
                    for j in range(D_HALF // LANES):
                        sl = pl.ds(j * LANES, LANES)
                        va = bufa[b, e, sl]
                        vb = bufb[b, e, sl]
                        lo = (lax.bitcast_convert_type(va << 16, jnp.float32)
                              + lax.bitcast_convert_type(vb << 16,
                                                         jnp.float32))
                        hi = (lax.bitcast_convert_type(va & M_HI, jnp.float32)
                              + lax.bitcast_convert_type(vb & M_HI,
                                                         jnp.float32))
                        lo_i = lax.shift_right_logical(
                            lax.bitcast_convert_type(lo, jnp.int32) + R_HALF,
                            16)
                        hi_i = (lax.bitcast_convert_type(hi, jnp.int32)
                                + R_HALF) & M_HI
                        bufa[b, e, sl] = lo_i | hi_i
                    return carry

                lax.fori_loop(0, CHUNK, row_body, 0)
                out_copy(chunk, b).start()

                nb = (b + PREF) % NBUF
                ib = (b + PREF + 1) % NBUF

                @pl.when(chunk + PREF + 1 < n_chunks)
                def _prefetch_idx():
                    idx_start(chunk + PREF + 1, ib)

                @pl.when((chunk + PREF < n_chunks) & (chunk >= NBUF - PREF))
                def _wait_writeout():
                    out_copy(chunk - (NBUF - PREF), nb).wait()

                @pl.when(chunk + PREF < n_chunks)
                def _prefetch_gathers():
                    idx_wait(chunk + PREF, nb)
                    gather_a(chunk + PREF, nb).start()
                    gather_b(chunk + PREF, nb).start()

        for b in range(NBUF):
            out_copy(n_chunks - NBUF + b, b).wait()

    return pl.kernel(
        body,
        out_type=jax.ShapeDtypeStruct((NW * epw, D_HALF), jnp.int32),
        mesh=plsc.VectorSubcoreMesh(core_axis_name="c", subcore_axis_name="s"),
        scratch_types=[
            pltpu.VMEM((NBUF, 2, CHUNK), jnp.int32),
            pltpu.VMEM((NBUF, CHUNK, D_HALF), jnp.int32),
            pltpu.VMEM((NBUF, CHUNK, D_HALF), jnp.int32),
            pltpu.SemaphoreType.DMA((NBUF,)),
            pltpu.SemaphoreType.DMA((NBUF,)),
            pltpu.SemaphoreType.DMA((NBUF,)),
            pltpu.SemaphoreType.DMA((NBUF,)),
        ],
    )


_gather_sum_1 = _make_gather_sum(0, CHUNKS_1)
_gather_sum_2 = _make_gather_sum(E_SLICE1, CHUNKS_2)


def _pack_halves(y):
    ylo = lax.bitcast_convert_type(y[:, :D_HALF], jnp.int32)
    yhi = lax.bitcast_convert_type(y[:, D_HALF:], jnp.int32)
    lo16 = lax.shift_right_logical(ylo + R_HALF, 16)
    hi16 = (yhi + R_HALF) & M_HI
    return lo16 | hi16


def _project_body(x_ref, ws_ref, wt_ref, xs_ref, xt_ref):
    xb = x_ref[...]
    xs_ref[...] = _pack_halves(
        jnp.dot(xb, ws_ref[...], preferred_element_type=jnp.float32))
    xt_ref[...] = _pack_halves(
        jnp.dot(xb, wt_ref[...], preferred_element_type=jnp.float32))


def _project(x, ws, wt):
    m_blk = 1000
    grid = (N_NODES // m_blk,)
    return pl.pallas_call(
        _project_body,
        grid=grid,
        in_specs=[
            pl.BlockSpec((m_blk, D_FEAT), lambda i: (i, 0)),
            pl.BlockSpec((D_FEAT, D_FEAT), lambda i: (0, 0)),
            pl.BlockSpec((D_FEAT, D_FEAT), lambda i: (0, 0)),
        ],
        out_specs=[
            pl.BlockSpec((m_blk, D_HALF), lambda i: (i, 0)),
            pl.BlockSpec((m_blk, D_HALF), lambda i: (i, 0)),
        ],
        out_shape=[
            jax.ShapeDtypeStruct((N_NODES, D_HALF), jnp.int32),
            jax.ShapeDtypeStruct((N_NODES, D_HALF), jnp.int32),
        ],
    )(x, ws, wt)


def _final_body(ef_ref, s_ref, we_ref, b_ref, o_ref):
    acc = jnp.dot(ef_ref[...], we_ref[...], preferred_element_type=jnp.float32)
    si = s_ref[...]
    lo = lax.bitcast_convert_type(si << 16, jnp.float32)
    hi = lax.bitcast_convert_type(si & M_HI, jnp.float32)
    s = jnp.concatenate([lo, hi], axis=1)
    o_ref[...] = jnp.maximum(acc + s + b_ref[...], 0.0)


def _final_slice1(edge_feat, s, we, b2d):
    grid = (E_SLICE1 // M_FIN,)
    return pl.pallas_call(
        _final_body,
        grid=grid,
        in_specs=[
            pl.BlockSpec((M_FIN, D_EDGE), lambda i: (i, 0)),
            pl.BlockSpec((M_FIN, D_HALF), lambda i: (i, 0)),
            pl.BlockSpec((D_EDGE, D_OUT), lambda i: (0, 0)),
            pl.BlockSpec((1, D_OUT), lambda i: (0, 0)),
        ],
        out_specs=pl.BlockSpec((M_FIN, D_OUT), lambda i: (i, 0)),
        out_shape=jax.ShapeDtypeStruct((N_EDGES, D_OUT), jnp.float32),
    )(edge_feat, s, we, b2d)


def _final_body2(_, ef_ref, s_ref, we_ref, b_ref, o_ref):
    _final_body(ef_ref, s_ref, we_ref, b_ref, o_ref)


def _final_slice2(out1, edge_feat, s, we, b2d):
    grid = (E_SLICE2 // M_FIN,)
    off = E_SLICE1 // M_FIN
    return pl.pallas_call(
        _final_body2,
        grid=grid,
        in_specs=[
            pl.BlockSpec(memory_space=pl.ANY),
            pl.BlockSpec((M_FIN, D_EDGE), lambda i: (i + off, 0)),
            pl.BlockSpec((M_FIN, D_HALF), lambda i: (i, 0)),
            pl.BlockSpec((D_EDGE, D_OUT), lambda i: (0, 0)),
            pl.BlockSpec((1, D_OUT), lambda i: (0, 0)),
        ],
        out_specs=pl.BlockSpec((M_FIN, D_OUT), lambda i: (i + off, 0)),
        out_shape=jax.ShapeDtypeStruct((N_EDGES, D_OUT), jnp.float32),
        input_output_aliases={0: 0},
    )(out1, edge_feat, s, we, b2d)


def kernel(x, edge_feat, edge_index, W, b):
    we = W[:D_EDGE]
    ws = W[D_EDGE:D_EDGE + D_FEAT]
    wt = W[D_EDGE + D_FEAT:]
    b2d = b.reshape(1, D_OUT)
    xs, xt = _project(x, ws, wt)
    iflat = edge_index.reshape(2 * N_EDGES)
    s1 = _gather_sum_1(xs, xt, iflat)
    s2 = _gather_sum_2(xs, xt, iflat)
    out1 = _final_slice1(edge_feat, s1, we, b2d)
    return _final_slice2(out1, edge_feat, s2, we, b2d)
